# baseline (speedup 1.0000x reference)
import functools

import jax
import jax.numpy as jnp
from jax import lax
from jax.experimental import pallas as pl
from jax.experimental.pallas import tpu as pltpu
from jax.experimental.pallas import tpu_sc as plsc

S = 2048
H = 2048
FF = 1024
E = 16
TOPK = 2
NGROUP = 4
GSZ = E // NGROUP
TOPK_GROUP = 2
SCALE = 1.0

BG = 256
BM = 128
NB = 48
PADDED = NB * BM
FFT = 512
NFT = FF // FFT

NC = 2
NS = 16
NW = NC * NS

GROWS = PADDED // NW
GCH = 16
NGC = GROWS // GCH
TPW = S // NW
CPAIR = 8
NCC = TPW // CPAIR
NEG_INF = float("-inf")


def _gate_body(x_ref, gw_ref, gb_ref, idx_ref, w_ref):
    x = x_ref[...]
    gw = gw_ref[...]
    logits = lax.dot_general(x, gw, (((1,), (1,)), ((), ())),
                             preferred_element_type=jnp.float32)
    scores = jax.nn.sigmoid(logits)
    sfc = scores + gb_ref[...]
    iota_e = lax.broadcasted_iota(jnp.int32, (BG, E), 1)
    group_id = iota_e // GSZ

    gscore_full = jnp.zeros((BG, E), jnp.float32)
    for k in range(NGROUP):
        in_g = group_id == k
        m1 = jnp.max(jnp.where(in_g, sfc, NEG_INF), axis=-1, keepdims=True)
        i1 = jnp.min(jnp.where(in_g & (sfc == m1), iota_e, E),
                     axis=-1, keepdims=True)
        m2 = jnp.max(jnp.where(in_g & (iota_e != i1), sfc, NEG_INF),
                     axis=-1, keepdims=True)
        gscore_full = jnp.where(in_g, m1 + m2, gscore_full)

    gm1 = jnp.max(gscore_full, axis=-1, keepdims=True)
    g1 = jnp.min(jnp.where(gscore_full == gm1, group_id, NGROUP),
                 axis=-1, keepdims=True)
    gs2 = jnp.where(group_id == g1, NEG_INF, gscore_full)
    gm2 = jnp.max(gs2, axis=-1, keepdims=True)
    g2 = jnp.min(jnp.where(gs2 == gm2, group_id, NGROUP),
                 axis=-1, keepdims=True)
    group_sel = (group_id == g1) | (group_id == g2)

    tmp = jnp.where(group_sel, sfc, NEG_INF)
    t1 = jnp.max(tmp, axis=-1, keepdims=True)
    e1 = jnp.min(jnp.where(tmp == t1, iota_e, E), axis=-1, keepdims=True)
    tmp2 = jnp.where(iota_e == e1, NEG_INF, tmp)
    t2 = jnp.max(tmp2, axis=-1, keepdims=True)
    e2 = jnp.min(jnp.where(tmp2 == t2, iota_e, E), axis=-1, keepdims=True)

    w1 = jnp.sum(jnp.where(iota_e == e1, scores, 0.0), axis=-1, keepdims=True)
    w2 = jnp.sum(jnp.where(iota_e == e2, scores, 0.0), axis=-1, keepdims=True)
    denom = w1 + w2 + 1e-20
    idx_ref[...] = jnp.concatenate([e1, e2], axis=1)
    w_ref[...] = (jnp.concatenate([w1, w2], axis=1) / denom) * SCALE


def _gate_call(hs, gate_weight, gate_bias):
    return pl.pallas_call(
        _gate_body,
        grid=(S // BG,),
        in_specs=[
            pl.BlockSpec((BG, H), lambda i: (i, 0)),
            pl.BlockSpec((E, H), lambda i: (0, 0)),
            pl.BlockSpec((1, E), lambda i: (0, 0)),
        ],
        out_specs=[
            pl.BlockSpec((BG, TOPK), lambda i: (i, 0)),
            pl.BlockSpec((BG, TOPK), lambda i: (i, 0)),
        ],
        out_shape=[
            jax.ShapeDtypeStruct((S, TOPK), jnp.int32),
            jax.ShapeDtypeStruct((S, TOPK), jnp.float32),
        ],
    )(hs, gate_weight, gate_bias.reshape(1, E))


def _routing(topk_idx, topk_w):
    eflat = topk_idx.reshape(-1)
    order = jnp.argsort(eflat, stable=True)
    e_sorted = eflat[order]
    tok_sorted = (order // TOPK).astype(jnp.int32)
    w_sorted = topk_w.reshape(-1)[order]
    counts = jnp.bincount(eflat, length=E)
    bpe = (counts + BM - 1) // BM
    binc = jnp.cumsum(bpe)
    bstart = jnp.concatenate([jnp.zeros((1,), binc.dtype), binc[:-1]])
    nvb = binc[-1].astype(jnp.int32)
    cstart = jnp.concatenate(
        [jnp.zeros((1,), counts.dtype), jnp.cumsum(counts)[:-1]])
    rank = jnp.arange(S * TOPK, dtype=jnp.int32) - cstart[e_sorted]
    ppos = (bstart[e_sorted] * BM + rank).astype(jnp.int32)
    pad_tok = jnp.arange(PADDED, dtype=jnp.int32) % S
    gather_tok = pad_tok.at[ppos].set(tok_sorted)
    w_pad = jnp.zeros((PADDED,), jnp.float32).at[ppos].set(w_sorted)
    blocks = jnp.arange(NB, dtype=jnp.int32)
    be_raw = jnp.searchsorted(binc, blocks, side="right").astype(jnp.int32)
    be = jnp.where(blocks < nvb, be_raw, be_raw[nvb - 1])
    bxi = jnp.minimum(blocks, nvb - 1)
    inv = jnp.zeros((S * TOPK,), jnp.int32).at[order].set(ppos)
    nv_rows = nvb * BM
    nv_arr = jnp.zeros((16,), jnp.int32) + nv_rows
    return (gather_tok, w_pad.reshape(NB, BM, 1), be, bxi,
            jnp.reshape(nvb, (1,)), nv_arr, inv)


def _sc_gather_body(hs_hbm, gtok_hbm, nv_hbm, out_hbm,
                    idx_v, rows_a, rows_b, nv_v, sem_a, sem_b):
    wid = lax.axis_index("s") * NC + lax.axis_index("c")
    base = wid * GROWS
    pltpu.sync_copy(nv_hbm.at[pl.ds(0, 16)], nv_v)
    nv = nv_v[...][0]

    @pl.when(base < nv)
    def _():
        pltpu.sync_copy(gtok_hbm.at[pl.ds(base, GROWS)], idx_v)
        bufs = (rows_a, rows_b)
        sems = (sem_a, sem_b)

        def fire(c):
            return pltpu.async_copy(
                hs_hbm.at[idx_v.at[pl.ds(c * GCH, GCH)]],
                bufs[c % 2], sems[c % 2])

        h = fire(0)
        for c in range(NGC):
            h_next = fire(c + 1) if c + 1 < NGC else None
            h.wait()
            pltpu.sync_copy(bufs[c % 2], out_hbm.at[pl.ds(base + c * GCH, GCH)])
            h = h_next


def _sc_gather_call(hs, gather_tok, nv_arr):
    mesh = plsc.VectorSubcoreMesh(core_axis_name="c", subcore_axis_name="s",
                                  num_cores=NC, num_subcores=NS)
    f = functools.partial(
        pl.kernel, _sc_gather_body, mesh=mesh,
        out_type=jax.ShapeDtypeStruct((PADDED, H), jnp.float32),
        scratch_types=[
            pltpu.VMEM((GROWS,), jnp.int32),
            pltpu.VMEM((GCH, H), jnp.float32),
            pltpu.VMEM((GCH, H), jnp.float32),
            pltpu.VMEM((16,), jnp.int32),
            pltpu.SemaphoreType.DMA,
            pltpu.SemaphoreType.DMA,
        ],
        name="sc_dispatch_gather",
    )()
    return f(hs, gather_tok, nv_arr)


def _ffn_body(nvb_ref, be_ref, bxi_ref, x_ref, wg_ref, wu_ref, wd_ref,
              wrow_ref, o_ref):
    b = pl.program_id(0)

    @pl.when(b < nvb_ref[0])
    def _():
        x = x_ref[...].astype(jnp.bfloat16)
        a = lax.dot_general(x, wg_ref[0], (((1,), (0,)), ((), ())),
                            preferred_element_type=jnp.float32)
        u = lax.dot_general(x, wu_ref[0], (((1,), (0,)), ((), ())),
                            preferred_element_type=jnp.float32)
        hblk = (a * jax.nn.sigmoid(a)) * u * wrow_ref[0]
        o_ref[...] = lax.dot_general(hblk.astype(jnp.bfloat16), wd_ref[0],
                                     (((1,), (0,)), ((), ())),
                                     preferred_element_type=jnp.float32)


def _ffn_call(x_sorted, w_gate, w_up, w_down, w_pad3, nvb, be, bxi):
    grid_spec = pltpu.PrefetchScalarGridSpec(
        num_scalar_prefetch=3,
        grid=(NB,),
        in_specs=[
            pl.BlockSpec((BM, H), lambda b, nvb, be, bxi: (bxi[b], 0)),
            pl.BlockSpec((1, H, FF), lambda b, nvb, be, bxi: (be[b], 0, 0)),
            pl.BlockSpec((1, H, FF), lambda b, nvb, be, bxi: (be[b], 0, 0)),
            pl.BlockSpec((1, FF, H), lambda b, nvb, be, bxi: (be[b], 0, 0)),
            pl.BlockSpec((1, BM, 1), lambda b, nvb, be, bxi: (b, 0, 0)),
        ],
        out_specs=pl.BlockSpec((BM, H), lambda b, nvb, be, bxi: (b, 0)),
    )
    return pl.pallas_call(
        _ffn_body,
        grid_spec=grid_spec,
        out_shape=jax.ShapeDtypeStruct((PADDED, H), jnp.float32),
    )(nvb, be, bxi, x_sorted, w_gate, w_up, w_down, w_pad3)


def _sc_combine_body(osort_hbm, inv_hbm, pairs_hbm,
                     idx_v, rows_a, rows_b, sem_a, sem_b):
    wid = lax.axis_index("s") * NC + lax.axis_index("c")
    base = wid * TPW * TOPK
    rows = TPW * TOPK
    pltpu.sync_copy(inv_hbm.at[pl.ds(base, rows)], idx_v)
    bufs = (rows_a, rows_b)
    sems = (sem_a, sem_b)
    rch = CPAIR * TOPK

    def fire(c):
        return pltpu.async_copy(
            osort_hbm.at[idx_v.at[pl.ds(c * rch, rch)]],
            bufs[c % 2], sems[c % 2])

    h = fire(0)
    for c in range(NCC):
        h_next = fire(c + 1) if c + 1 < NCC else None
        h.wait()
        pltpu.sync_copy(bufs[c % 2], pairs_hbm.at[pl.ds(base + c * rch, rch)])
        h = h_next


def _sc_combine_call(osort, inv):
    mesh = plsc.VectorSubcoreMesh(core_axis_name="c", subcore_axis_name="s",
                                  num_cores=NC, num_subcores=NS)
    f = functools.partial(
        pl.kernel, _sc_combine_body, mesh=mesh,
        out_type=jax.ShapeDtypeStruct((S * TOPK, H), jnp.float32),
        scratch_types=[
            pltpu.VMEM((TPW * TOPK,), jnp.int32),
            pltpu.VMEM((CPAIR * TOPK, H), jnp.float32),
            pltpu.VMEM((CPAIR * TOPK, H), jnp.float32),
            pltpu.SemaphoreType.DMA,
            pltpu.SemaphoreType.DMA,
        ],
        name="sc_combine_gather",
    )()
    return f(osort, inv)


def _add_body(a_ref, o_ref):
    o_ref[...] = a_ref[:, 0, :] + a_ref[:, 1, :]


def _add_call(pairs):
    return pl.pallas_call(
        _add_body,
        grid=(S // BG,),
        in_specs=[pl.BlockSpec((BG, TOPK, H), lambda i: (i, 0, 0))],
        out_specs=pl.BlockSpec((BG, H), lambda i: (i, 0)),
        out_shape=jax.ShapeDtypeStruct((S, H), jnp.float32),
    )(pairs.reshape(S, TOPK, H))


def kernel(hidden_states, gate_weight, gate_bias, w_gate, w_up, w_down):
    B_, S_, H_ = hidden_states.shape
    hs = hidden_states.reshape(S_, H_)
    topk_idx, topk_w = _gate_call(hs, gate_weight, gate_bias)
    (gather_tok, w_pad3, be, bxi, nvb, nv_arr, inv) = _routing(
        topk_idx, topk_w)
    x_sorted = _sc_gather_call(hs, gather_tok, nv_arr)
    osort = _ffn_call(x_sorted, w_gate.astype(jnp.bfloat16),
                      w_up.astype(jnp.bfloat16), w_down.astype(jnp.bfloat16),
                      w_pad3, nvb, be, bxi)
    pairs = _sc_combine_call(osort, inv)
    final = _add_call(pairs)
    return final.reshape(B_, S_, H_)


# speedup vs baseline: 1.3436x; 1.3436x over previous
import functools

import jax
import jax.numpy as jnp
from jax import lax
from jax.experimental import pallas as pl
from jax.experimental.pallas import tpu as pltpu
from jax.experimental.pallas import tpu_sc as plsc

S = 2048
H = 2048
FF = 1024
E = 16
TOPK = 2
NGROUP = 4
GSZ = E // NGROUP
TOPK_GROUP = 2
SCALE = 1.0

BG = 256
BM = 128
NB = 48
PADDED = NB * BM
FFT = 512
NFT = FF // FFT

NC = 2
NS = 16
NW = NC * NS

GROWS = PADDED // NW
GCH = 16
NGC = GROWS // GCH
TPW = S // NW
CPAIR = 8
NCC = TPW // CPAIR
NEG_INF = float("-inf")


def _gate_body(x_ref, gw_ref, gb_ref, idx_ref, w_ref):
    x = x_ref[...]
    gw = gw_ref[...]
    logits = lax.dot_general(x, gw, (((1,), (1,)), ((), ())),
                             preferred_element_type=jnp.float32)
    scores = jax.nn.sigmoid(logits)
    sfc = scores + gb_ref[...]
    iota_e = lax.broadcasted_iota(jnp.int32, (BG, E), 1)
    group_id = iota_e // GSZ

    gscore_full = jnp.zeros((BG, E), jnp.float32)
    for k in range(NGROUP):
        in_g = group_id == k
        m1 = jnp.max(jnp.where(in_g, sfc, NEG_INF), axis=-1, keepdims=True)
        i1 = jnp.min(jnp.where(in_g & (sfc == m1), iota_e, E),
                     axis=-1, keepdims=True)
        m2 = jnp.max(jnp.where(in_g & (iota_e != i1), sfc, NEG_INF),
                     axis=-1, keepdims=True)
        gscore_full = jnp.where(in_g, m1 + m2, gscore_full)

    gm1 = jnp.max(gscore_full, axis=-1, keepdims=True)
    g1 = jnp.min(jnp.where(gscore_full == gm1, group_id, NGROUP),
                 axis=-1, keepdims=True)
    gs2 = jnp.where(group_id == g1, NEG_INF, gscore_full)
    gm2 = jnp.max(gs2, axis=-1, keepdims=True)
    g2 = jnp.min(jnp.where(gs2 == gm2, group_id, NGROUP),
                 axis=-1, keepdims=True)
    group_sel = (group_id == g1) | (group_id == g2)

    tmp = jnp.where(group_sel, sfc, NEG_INF)
    t1 = jnp.max(tmp, axis=-1, keepdims=True)
    e1 = jnp.min(jnp.where(tmp == t1, iota_e, E), axis=-1, keepdims=True)
    tmp2 = jnp.where(iota_e == e1, NEG_INF, tmp)
    t2 = jnp.max(tmp2, axis=-1, keepdims=True)
    e2 = jnp.min(jnp.where(tmp2 == t2, iota_e, E), axis=-1, keepdims=True)

    w1 = jnp.sum(jnp.where(iota_e == e1, scores, 0.0), axis=-1, keepdims=True)
    w2 = jnp.sum(jnp.where(iota_e == e2, scores, 0.0), axis=-1, keepdims=True)
    denom = w1 + w2 + 1e-20
    idx_ref[...] = jnp.concatenate([e1, e2], axis=1)
    w_ref[...] = (jnp.concatenate([w1, w2], axis=1) / denom) * SCALE


def _gate_call(hs, gate_weight, gate_bias):
    return pl.pallas_call(
        _gate_body,
        grid=(S // BG,),
        in_specs=[
            pl.BlockSpec((BG, H), lambda i: (i, 0)),
            pl.BlockSpec((E, H), lambda i: (0, 0)),
            pl.BlockSpec((1, E), lambda i: (0, 0)),
        ],
        out_specs=[
            pl.BlockSpec((BG, TOPK), lambda i: (i, 0)),
            pl.BlockSpec((BG, TOPK), lambda i: (i, 0)),
        ],
        out_shape=[
            jax.ShapeDtypeStruct((S, TOPK), jnp.int32),
            jax.ShapeDtypeStruct((S, TOPK), jnp.float32),
        ],
    )(hs, gate_weight, gate_bias.reshape(1, E))


def _routing(topk_idx, topk_w):
    eflat = topk_idx.reshape(-1)
    order = jnp.argsort(eflat, stable=True)
    e_sorted = eflat[order]
    tok_sorted = (order // TOPK).astype(jnp.int32)
    w_sorted = topk_w.reshape(-1)[order]
    counts = jnp.bincount(eflat, length=E)
    bpe = (counts + BM - 1) // BM
    binc = jnp.cumsum(bpe)
    bstart = jnp.concatenate([jnp.zeros((1,), binc.dtype), binc[:-1]])
    nvb = binc[-1].astype(jnp.int32)
    cstart = jnp.concatenate(
        [jnp.zeros((1,), counts.dtype), jnp.cumsum(counts)[:-1]])
    rank = jnp.arange(S * TOPK, dtype=jnp.int32) - cstart[e_sorted]
    ppos = (bstart[e_sorted] * BM + rank).astype(jnp.int32)
    pad_tok = jnp.arange(PADDED, dtype=jnp.int32) % S
    gather_tok = pad_tok.at[ppos].set(tok_sorted)
    w_pad = jnp.zeros((PADDED,), jnp.float32).at[ppos].set(w_sorted)
    blocks = jnp.arange(NB, dtype=jnp.int32)
    be_raw = jnp.searchsorted(binc, blocks, side="right").astype(jnp.int32)
    be = jnp.where(blocks < nvb, be_raw, be_raw[nvb - 1])
    bxi = jnp.minimum(blocks, nvb - 1)
    inv = jnp.zeros((S * TOPK,), jnp.int32).at[order].set(ppos)
    nv_rows = nvb * BM
    nv_arr = jnp.zeros((16,), jnp.int32) + nv_rows
    return (gather_tok, w_pad.reshape(NB, BM, 1), be, bxi,
            jnp.reshape(nvb, (1,)), nv_arr, inv)


def _sc_gather_body(hs_hbm, gtok_hbm, nv_hbm, out_hbm,
                    idx_v, rows_a, rows_b, nv_v, sem_a, sem_b):
    wid = lax.axis_index("s") * NC + lax.axis_index("c")
    base = wid * GROWS
    pltpu.sync_copy(nv_hbm.at[pl.ds(0, 16)], nv_v)
    nv = nv_v[...][0]

    @pl.when(base < nv)
    def _():
        pltpu.sync_copy(gtok_hbm.at[pl.ds(base, GROWS)], idx_v)
        bufs = (rows_a, rows_b)
        sems = (sem_a, sem_b)

        def fire(c):
            return pltpu.async_copy(
                hs_hbm.at[idx_v.at[pl.ds(c * GCH, GCH)]],
                bufs[c % 2], sems[c % 2])

        h = fire(0)
        for c in range(NGC):
            h_next = fire(c + 1) if c + 1 < NGC else None
            h.wait()
            pltpu.sync_copy(bufs[c % 2], out_hbm.at[pl.ds(base + c * GCH, GCH)])
            h = h_next


def _sc_gather_call(hs, gather_tok, nv_arr):
    mesh = plsc.VectorSubcoreMesh(core_axis_name="c", subcore_axis_name="s",
                                  num_cores=NC, num_subcores=NS)
    f = functools.partial(
        pl.kernel, _sc_gather_body, mesh=mesh,
        out_type=jax.ShapeDtypeStruct((PADDED, H), jnp.float32),
        scratch_types=[
            pltpu.VMEM((GROWS,), jnp.int32),
            pltpu.VMEM((GCH, H), jnp.float32),
            pltpu.VMEM((GCH, H), jnp.float32),
            pltpu.VMEM((16,), jnp.int32),
            pltpu.SemaphoreType.DMA,
            pltpu.SemaphoreType.DMA,
        ],
        name="sc_dispatch_gather",
    )()
    return f(hs, gather_tok, nv_arr)


def _ffn_body(nvb_ref, be_ref, bxi_ref, x_ref, wg_ref, wu_ref, wd_ref,
              wrow_ref, o_ref):
    b = pl.program_id(0)

    @pl.when(b < nvb_ref[0])
    def _():
        x = x_ref[...]
        a = lax.dot_general(x, wg_ref[0], (((1,), (0,)), ((), ())),
                            preferred_element_type=jnp.float32)
        u = lax.dot_general(x, wu_ref[0], (((1,), (0,)), ((), ())),
                            preferred_element_type=jnp.float32)
        hblk = (a * jax.nn.sigmoid(a)) * u * wrow_ref[0]
        o_ref[...] = lax.dot_general(hblk, wd_ref[0], (((1,), (0,)), ((), ())),
                                     preferred_element_type=jnp.float32)


def _ffn_call(x_sorted, w_gate, w_up, w_down, w_pad3, nvb, be, bxi):
    grid_spec = pltpu.PrefetchScalarGridSpec(
        num_scalar_prefetch=3,
        grid=(NB,),
        in_specs=[
            pl.BlockSpec((BM, H), lambda b, nvb, be, bxi: (bxi[b], 0)),
            pl.BlockSpec((1, H, FF), lambda b, nvb, be, bxi: (be[b], 0, 0)),
            pl.BlockSpec((1, H, FF), lambda b, nvb, be, bxi: (be[b], 0, 0)),
            pl.BlockSpec((1, FF, H), lambda b, nvb, be, bxi: (be[b], 0, 0)),
            pl.BlockSpec((1, BM, 1), lambda b, nvb, be, bxi: (b, 0, 0)),
        ],
        out_specs=pl.BlockSpec((BM, H), lambda b, nvb, be, bxi: (b, 0)),
    )
    return pl.pallas_call(
        _ffn_body,
        grid_spec=grid_spec,
        out_shape=jax.ShapeDtypeStruct((PADDED, H), jnp.float32),
    )(nvb, be, bxi, x_sorted, w_gate, w_up, w_down, w_pad3)


def _sc_combine_body(osort_hbm, inv_hbm, pairs_hbm,
                     idx_v, rows_a, rows_b, sem_a, sem_b):
    wid = lax.axis_index("s") * NC + lax.axis_index("c")
    base = wid * TPW * TOPK
    rows = TPW * TOPK
    pltpu.sync_copy(inv_hbm.at[pl.ds(base, rows)], idx_v)
    bufs = (rows_a, rows_b)
    sems = (sem_a, sem_b)
    rch = CPAIR * TOPK

    def fire(c):
        return pltpu.async_copy(
            osort_hbm.at[idx_v.at[pl.ds(c * rch, rch)]],
            bufs[c % 2], sems[c % 2])

    h = fire(0)
    for c in range(NCC):
        h_next = fire(c + 1) if c + 1 < NCC else None
        h.wait()
        pltpu.sync_copy(bufs[c % 2], pairs_hbm.at[pl.ds(base + c * rch, rch)])
        h = h_next


def _sc_combine_call(osort, inv):
    mesh = plsc.VectorSubcoreMesh(core_axis_name="c", subcore_axis_name="s",
                                  num_cores=NC, num_subcores=NS)
    f = functools.partial(
        pl.kernel, _sc_combine_body, mesh=mesh,
        out_type=jax.ShapeDtypeStruct((S * TOPK, H), jnp.float32),
        scratch_types=[
            pltpu.VMEM((TPW * TOPK,), jnp.int32),
            pltpu.VMEM((CPAIR * TOPK, H), jnp.float32),
            pltpu.VMEM((CPAIR * TOPK, H), jnp.float32),
            pltpu.SemaphoreType.DMA,
            pltpu.SemaphoreType.DMA,
        ],
        name="sc_combine_gather",
    )()
    return f(osort, inv)


def _add_body(a_ref, o_ref):
    o_ref[...] = a_ref[:, 0, :] + a_ref[:, 1, :]


def _add_call(pairs):
    return pl.pallas_call(
        _add_body,
        grid=(S // BG,),
        in_specs=[pl.BlockSpec((BG, TOPK, H), lambda i: (i, 0, 0))],
        out_specs=pl.BlockSpec((BG, H), lambda i: (i, 0)),
        out_shape=jax.ShapeDtypeStruct((S, H), jnp.float32),
    )(pairs.reshape(S, TOPK, H))


def kernel(hidden_states, gate_weight, gate_bias, w_gate, w_up, w_down):
    B_, S_, H_ = hidden_states.shape
    hs = hidden_states.reshape(S_, H_)
    topk_idx, topk_w = _gate_call(hs, gate_weight, gate_bias)
    (gather_tok, w_pad3, be, bxi, nvb, nv_arr, inv) = _routing(
        topk_idx, topk_w)
    x_sorted = _sc_gather_call(hs, gather_tok, nv_arr)
    osort = _ffn_call(x_sorted, w_gate, w_up, w_down, w_pad3, nvb, be, bxi)
    pairs = _sc_combine_call(osort, inv)
    final = _add_call(pairs)
    return final.reshape(B_, S_, H_)


# speedup vs baseline: 1.5030x; 1.1186x over previous
import functools

import jax
import jax.numpy as jnp
from jax import lax
from jax.experimental import pallas as pl
from jax.experimental.pallas import tpu as pltpu
from jax.experimental.pallas import tpu_sc as plsc

S = 2048
H = 2048
FF = 1024
E = 16
TOPK = 2
NGROUP = 4
GSZ = E // NGROUP
TOPK_GROUP = 2
SCALE = 1.0

BG = 256
BM = 128
NB = 48
PADDED = NB * BM
FFT = 512
NFT = FF // FFT

NC = 2
NS = 16
NW = NC * NS

GROWS = PADDED // NW
GCH = 16
NGC = GROWS // GCH
TPW = S // NW
CPAIR = 8
NCC = TPW // CPAIR
NEG_INF = float("-inf")


def _gate_body(x_ref, gw_ref, gb_ref, idx_ref, w_ref):
    x = x_ref[...]
    gw = gw_ref[...]
    logits = lax.dot_general(x, gw, (((1,), (1,)), ((), ())),
                             preferred_element_type=jnp.float32)
    scores = jax.nn.sigmoid(logits)
    sfc = scores + gb_ref[...]
    iota_e = lax.broadcasted_iota(jnp.int32, (BG, E), 1)
    group_id = iota_e // GSZ

    gscore_full = jnp.zeros((BG, E), jnp.float32)
    for k in range(NGROUP):
        in_g = group_id == k
        m1 = jnp.max(jnp.where(in_g, sfc, NEG_INF), axis=-1, keepdims=True)
        i1 = jnp.min(jnp.where(in_g & (sfc == m1), iota_e, E),
                     axis=-1, keepdims=True)
        m2 = jnp.max(jnp.where(in_g & (iota_e != i1), sfc, NEG_INF),
                     axis=-1, keepdims=True)
        gscore_full = jnp.where(in_g, m1 + m2, gscore_full)

    gm1 = jnp.max(gscore_full, axis=-1, keepdims=True)
    g1 = jnp.min(jnp.where(gscore_full == gm1, group_id, NGROUP),
                 axis=-1, keepdims=True)
    gs2 = jnp.where(group_id == g1, NEG_INF, gscore_full)
    gm2 = jnp.max(gs2, axis=-1, keepdims=True)
    g2 = jnp.min(jnp.where(gs2 == gm2, group_id, NGROUP),
                 axis=-1, keepdims=True)
    group_sel = (group_id == g1) | (group_id == g2)

    tmp = jnp.where(group_sel, sfc, NEG_INF)
    t1 = jnp.max(tmp, axis=-1, keepdims=True)
    e1 = jnp.min(jnp.where(tmp == t1, iota_e, E), axis=-1, keepdims=True)
    tmp2 = jnp.where(iota_e == e1, NEG_INF, tmp)
    t2 = jnp.max(tmp2, axis=-1, keepdims=True)
    e2 = jnp.min(jnp.where(tmp2 == t2, iota_e, E), axis=-1, keepdims=True)

    w1 = jnp.sum(jnp.where(iota_e == e1, scores, 0.0), axis=-1, keepdims=True)
    w2 = jnp.sum(jnp.where(iota_e == e2, scores, 0.0), axis=-1, keepdims=True)
    denom = w1 + w2 + 1e-20
    idx_ref[...] = jnp.concatenate([e1, e2], axis=1)
    w_ref[...] = (jnp.concatenate([w1, w2], axis=1) / denom) * SCALE


def _gate_call(hs, gate_weight, gate_bias):
    return pl.pallas_call(
        _gate_body,
        grid=(S // BG,),
        in_specs=[
            pl.BlockSpec((BG, H), lambda i: (i, 0)),
            pl.BlockSpec((E, H), lambda i: (0, 0)),
            pl.BlockSpec((1, E), lambda i: (0, 0)),
        ],
        out_specs=[
            pl.BlockSpec((BG, TOPK), lambda i: (i, 0)),
            pl.BlockSpec((BG, TOPK), lambda i: (i, 0)),
        ],
        out_shape=[
            jax.ShapeDtypeStruct((S, TOPK), jnp.int32),
            jax.ShapeDtypeStruct((S, TOPK), jnp.float32),
        ],
    )(hs, gate_weight, gate_bias.reshape(1, E))


def _routing(topk_idx, topk_w):
    npair = S * TOPK
    eflat = topk_idx.reshape(-1)
    pair_id = jnp.arange(npair, dtype=jnp.int32)
    e_sorted, order, w_sorted = lax.sort(
        [eflat, pair_id, topk_w.reshape(-1)], num_keys=1, is_stable=True)
    tok_sorted = order // TOPK
    oh = (e_sorted[:, None] == jnp.arange(E, dtype=jnp.int32)[None, :])
    ohf = oh.astype(jnp.float32)
    counts_f = jnp.sum(ohf, axis=0)
    bpe_f = jnp.ceil(counts_f / BM)
    binc_f = jnp.cumsum(bpe_f)
    bstart_f = binc_f - bpe_f
    cstart_f = jnp.cumsum(counts_f) - counts_f
    cst = ohf @ cstart_f
    bst = ohf @ bstart_f
    rank = jnp.arange(npair, dtype=jnp.float32) - cst
    ppos = (bst * BM + rank).astype(jnp.int32)
    nvb = binc_f[-1].astype(jnp.int32)
    pad_tok = jnp.arange(PADDED, dtype=jnp.int32) % S
    gather_tok = pad_tok.at[ppos].set(tok_sorted)
    w_pad = jnp.zeros((PADDED,), jnp.float32).at[ppos].set(w_sorted)
    blocks = jnp.arange(NB, dtype=jnp.int32)
    binc = binc_f.astype(jnp.int32)
    be_raw = jnp.sum((binc[None, :] <= blocks[:, None]).astype(jnp.int32),
                     axis=1)
    counts_i = counts_f.astype(jnp.int32)
    e_last = jnp.max(jnp.where(counts_i > 0, jnp.arange(E, dtype=jnp.int32), -1))
    be = jnp.where(blocks < nvb, be_raw, e_last)
    bxi = jnp.minimum(blocks, nvb - 1)
    _, inv = lax.sort([order, ppos], num_keys=1)
    nv_rows = nvb * BM
    nv_arr = jnp.zeros((16,), jnp.int32) + nv_rows
    return (gather_tok, w_pad.reshape(NB, BM, 1), be, bxi,
            jnp.reshape(nvb, (1,)), nv_arr, inv)


def _sc_gather_body(hs_hbm, gtok_hbm, nv_hbm, out_hbm,
                    idx_v, rows_a, rows_b, nv_v, sem_a, sem_b):
    wid = lax.axis_index("s") * NC + lax.axis_index("c")
    base = wid * GROWS
    pltpu.sync_copy(nv_hbm.at[pl.ds(0, 16)], nv_v)
    nv = nv_v[...][0]

    @pl.when(base < nv)
    def _():
        pltpu.sync_copy(gtok_hbm.at[pl.ds(base, GROWS)], idx_v)
        bufs = (rows_a, rows_b)
        sems = (sem_a, sem_b)

        def fire(c):
            return pltpu.async_copy(
                hs_hbm.at[idx_v.at[pl.ds(c * GCH, GCH)]],
                bufs[c % 2], sems[c % 2])

        h = fire(0)
        for c in range(NGC):
            h_next = fire(c + 1) if c + 1 < NGC else None
            h.wait()
            pltpu.sync_copy(bufs[c % 2], out_hbm.at[pl.ds(base + c * GCH, GCH)])
            h = h_next


def _sc_gather_call(hs, gather_tok, nv_arr):
    mesh = plsc.VectorSubcoreMesh(core_axis_name="c", subcore_axis_name="s",
                                  num_cores=NC, num_subcores=NS)
    f = functools.partial(
        pl.kernel, _sc_gather_body, mesh=mesh,
        out_type=jax.ShapeDtypeStruct((PADDED, H), jnp.float32),
        scratch_types=[
            pltpu.VMEM((GROWS,), jnp.int32),
            pltpu.VMEM((GCH, H), jnp.float32),
            pltpu.VMEM((GCH, H), jnp.float32),
            pltpu.VMEM((16,), jnp.int32),
            pltpu.SemaphoreType.DMA,
            pltpu.SemaphoreType.DMA,
        ],
        name="sc_dispatch_gather",
    )()
    return f(hs, gather_tok, nv_arr)


def _ffn_body(nvb_ref, be_ref, bxi_ref, x_ref, wg_ref, wu_ref, wd_ref,
              wrow_ref, o_ref):
    b = pl.program_id(0)

    @pl.when(b < nvb_ref[0])
    def _():
        x = x_ref[...]
        a = lax.dot_general(x, wg_ref[0], (((1,), (0,)), ((), ())),
                            preferred_element_type=jnp.float32)
        u = lax.dot_general(x, wu_ref[0], (((1,), (0,)), ((), ())),
                            preferred_element_type=jnp.float32)
        hblk = (a * jax.nn.sigmoid(a)) * u * wrow_ref[0]
        o_ref[...] = lax.dot_general(hblk, wd_ref[0], (((1,), (0,)), ((), ())),
                                     preferred_element_type=jnp.float32)


def _ffn_call(x_sorted, w_gate, w_up, w_down, w_pad3, nvb, be, bxi):
    grid_spec = pltpu.PrefetchScalarGridSpec(
        num_scalar_prefetch=3,
        grid=(NB,),
        in_specs=[
            pl.BlockSpec((BM, H), lambda b, nvb, be, bxi: (bxi[b], 0)),
            pl.BlockSpec((1, H, FF), lambda b, nvb, be, bxi: (be[b], 0, 0)),
            pl.BlockSpec((1, H, FF), lambda b, nvb, be, bxi: (be[b], 0, 0)),
            pl.BlockSpec((1, FF, H), lambda b, nvb, be, bxi: (be[b], 0, 0)),
            pl.BlockSpec((1, BM, 1), lambda b, nvb, be, bxi: (b, 0, 0)),
        ],
        out_specs=pl.BlockSpec((BM, H), lambda b, nvb, be, bxi: (b, 0)),
    )
    return pl.pallas_call(
        _ffn_body,
        grid_spec=grid_spec,
        out_shape=jax.ShapeDtypeStruct((PADDED, H), jnp.float32),
    )(nvb, be, bxi, x_sorted, w_gate, w_up, w_down, w_pad3)


def _sc_combine_body(osort_hbm, inv_hbm, pairs_hbm,
                     idx_v, rows_a, rows_b, sem_a, sem_b):
    wid = lax.axis_index("s") * NC + lax.axis_index("c")
    base = wid * TPW * TOPK
    rows = TPW * TOPK
    pltpu.sync_copy(inv_hbm.at[pl.ds(base, rows)], idx_v)
    bufs = (rows_a, rows_b)
    sems = (sem_a, sem_b)
    rch = CPAIR * TOPK

    def fire(c):
        return pltpu.async_copy(
            osort_hbm.at[idx_v.at[pl.ds(c * rch, rch)]],
            bufs[c % 2], sems[c % 2])

    h = fire(0)
    for c in range(NCC):
        h_next = fire(c + 1) if c + 1 < NCC else None
        h.wait()
        pltpu.sync_copy(bufs[c % 2], pairs_hbm.at[pl.ds(base + c * rch, rch)])
        h = h_next


def _sc_combine_call(osort, inv):
    mesh = plsc.VectorSubcoreMesh(core_axis_name="c", subcore_axis_name="s",
                                  num_cores=NC, num_subcores=NS)
    f = functools.partial(
        pl.kernel, _sc_combine_body, mesh=mesh,
        out_type=jax.ShapeDtypeStruct((S * TOPK, H), jnp.float32),
        scratch_types=[
            pltpu.VMEM((TPW * TOPK,), jnp.int32),
            pltpu.VMEM((CPAIR * TOPK, H), jnp.float32),
            pltpu.VMEM((CPAIR * TOPK, H), jnp.float32),
            pltpu.SemaphoreType.DMA,
            pltpu.SemaphoreType.DMA,
        ],
        name="sc_combine_gather",
    )()
    return f(osort, inv)


def _add_body(a_ref, o_ref):
    o_ref[...] = a_ref[:, 0, :] + a_ref[:, 1, :]


def _add_call(pairs):
    return pl.pallas_call(
        _add_body,
        grid=(S // BG,),
        in_specs=[pl.BlockSpec((BG, TOPK, H), lambda i: (i, 0, 0))],
        out_specs=pl.BlockSpec((BG, H), lambda i: (i, 0)),
        out_shape=jax.ShapeDtypeStruct((S, H), jnp.float32),
    )(pairs.reshape(S, TOPK, H))


def kernel(hidden_states, gate_weight, gate_bias, w_gate, w_up, w_down):
    B_, S_, H_ = hidden_states.shape
    hs = hidden_states.reshape(S_, H_)
    topk_idx, topk_w = _gate_call(hs, gate_weight, gate_bias)
    (gather_tok, w_pad3, be, bxi, nvb, nv_arr, inv) = _routing(
        topk_idx, topk_w)
    x_sorted = _sc_gather_call(hs, gather_tok, nv_arr)
    osort = _ffn_call(x_sorted, w_gate, w_up, w_down, w_pad3, nvb, be, bxi)
    pairs = _sc_combine_call(osort, inv)
    final = _add_call(pairs)
    return final.reshape(B_, S_, H_)


# speedup vs baseline: 1.5043x; 1.0008x over previous
import functools

import jax
import jax.numpy as jnp
from jax import lax
from jax.experimental import pallas as pl
from jax.experimental.pallas import tpu as pltpu
from jax.experimental.pallas import tpu_sc as plsc

S = 2048
H = 2048
FF = 1024
E = 16
TOPK = 2
NGROUP = 4
GSZ = E // NGROUP
TOPK_GROUP = 2
SCALE = 1.0

BG = 256
BM = 128
NB = 48
PADDED = NB * BM
FFT = 512
NFT = FF // FFT

NC = 2
NS = 16
NW = NC * NS

GROWS = PADDED // NW
GCH = 16
NGC = GROWS // GCH
TPW = S // NW
CPAIR = 8
NCC = TPW // CPAIR
NEG_INF = float("-inf")


def _gate_body(x_ref, gw_ref, gb_ref, idx_ref, w_ref):
    x = x_ref[...]
    gw = gw_ref[...]
    logits = lax.dot_general(x, gw, (((1,), (1,)), ((), ())),
                             preferred_element_type=jnp.float32)
    scores = jax.nn.sigmoid(logits)
    sfc = scores + gb_ref[...]
    iota_e = lax.broadcasted_iota(jnp.int32, (BG, E), 1)
    group_id = iota_e // GSZ

    gscore_full = jnp.zeros((BG, E), jnp.float32)
    for k in range(NGROUP):
        in_g = group_id == k
        m1 = jnp.max(jnp.where(in_g, sfc, NEG_INF), axis=-1, keepdims=True)
        i1 = jnp.min(jnp.where(in_g & (sfc == m1), iota_e, E),
                     axis=-1, keepdims=True)
        m2 = jnp.max(jnp.where(in_g & (iota_e != i1), sfc, NEG_INF),
                     axis=-1, keepdims=True)
        gscore_full = jnp.where(in_g, m1 + m2, gscore_full)

    gm1 = jnp.max(gscore_full, axis=-1, keepdims=True)
    g1 = jnp.min(jnp.where(gscore_full == gm1, group_id, NGROUP),
                 axis=-1, keepdims=True)
    gs2 = jnp.where(group_id == g1, NEG_INF, gscore_full)
    gm2 = jnp.max(gs2, axis=-1, keepdims=True)
    g2 = jnp.min(jnp.where(gs2 == gm2, group_id, NGROUP),
                 axis=-1, keepdims=True)
    group_sel = (group_id == g1) | (group_id == g2)

    tmp = jnp.where(group_sel, sfc, NEG_INF)
    t1 = jnp.max(tmp, axis=-1, keepdims=True)
    e1 = jnp.min(jnp.where(tmp == t1, iota_e, E), axis=-1, keepdims=True)
    tmp2 = jnp.where(iota_e == e1, NEG_INF, tmp)
    t2 = jnp.max(tmp2, axis=-1, keepdims=True)
    e2 = jnp.min(jnp.where(tmp2 == t2, iota_e, E), axis=-1, keepdims=True)

    w1 = jnp.sum(jnp.where(iota_e == e1, scores, 0.0), axis=-1, keepdims=True)
    w2 = jnp.sum(jnp.where(iota_e == e2, scores, 0.0), axis=-1, keepdims=True)
    denom = w1 + w2 + 1e-20
    idx_ref[...] = jnp.concatenate([e1, e2], axis=1)
    w_ref[...] = (jnp.concatenate([w1, w2], axis=1) / denom) * SCALE


def _gate_call(hs, gate_weight, gate_bias):
    return pl.pallas_call(
        _gate_body,
        grid=(S // BG,),
        in_specs=[
            pl.BlockSpec((BG, H), lambda i: (i, 0)),
            pl.BlockSpec((E, H), lambda i: (0, 0)),
            pl.BlockSpec((1, E), lambda i: (0, 0)),
        ],
        out_specs=[
            pl.BlockSpec((BG, TOPK), lambda i: (i, 0)),
            pl.BlockSpec((BG, TOPK), lambda i: (i, 0)),
        ],
        out_shape=[
            jax.ShapeDtypeStruct((S, TOPK), jnp.int32),
            jax.ShapeDtypeStruct((S, TOPK), jnp.float32),
        ],
    )(hs, gate_weight, gate_bias.reshape(1, E))


def _routing(topk_idx, topk_w):
    npair = S * TOPK
    eflat = topk_idx.reshape(-1)
    pair_id = jnp.arange(npair, dtype=jnp.int32)
    e_sorted, order, w_sorted = lax.sort(
        [eflat, pair_id, topk_w.reshape(-1)], num_keys=1, is_stable=True)
    tok_sorted = order // TOPK
    oh = (e_sorted[:, None] == jnp.arange(E, dtype=jnp.int32)[None, :])
    ohf = oh.astype(jnp.float32)
    counts_f = jnp.sum(ohf, axis=0)
    bpe_f = jnp.ceil(counts_f / BM)
    binc_f = jnp.cumsum(bpe_f)
    bstart_f = binc_f - bpe_f
    cstart_f = jnp.cumsum(counts_f) - counts_f
    cst = jnp.sum(jnp.where(oh, cstart_f[None, :], 0.0), axis=1)
    bst = jnp.sum(jnp.where(oh, bstart_f[None, :], 0.0), axis=1)
    rank = jnp.arange(npair, dtype=jnp.float32) - cst
    ppos = (bst * BM + rank).astype(jnp.int32)
    nvb = binc_f[-1].astype(jnp.int32)
    pad_tok = jnp.arange(PADDED, dtype=jnp.int32) % S
    gather_tok = pad_tok.at[ppos].set(tok_sorted)
    w_pad = jnp.zeros((PADDED,), jnp.float32).at[ppos].set(w_sorted)
    blocks = jnp.arange(NB, dtype=jnp.int32)
    binc = binc_f.astype(jnp.int32)
    be_raw = jnp.sum((binc[None, :] <= blocks[:, None]).astype(jnp.int32),
                     axis=1)
    counts_i = counts_f.astype(jnp.int32)
    e_last = jnp.max(jnp.where(counts_i > 0, jnp.arange(E, dtype=jnp.int32), -1))
    be = jnp.where(blocks < nvb, be_raw, e_last)
    bxi = jnp.minimum(blocks, nvb - 1)
    _, inv = lax.sort([order, ppos], num_keys=1)
    nv_rows = nvb * BM
    nv_arr = jnp.zeros((16,), jnp.int32) + nv_rows
    return (gather_tok, w_pad.reshape(NB, BM, 1), be, bxi,
            jnp.reshape(nvb, (1,)), nv_arr, inv)


def _sc_gather_body(hs_hbm, gtok_hbm, nv_hbm, out_hbm,
                    idx_v, rows_a, rows_b, nv_v, sem_a, sem_b):
    wid = lax.axis_index("s") * NC + lax.axis_index("c")
    base = wid * GROWS
    pltpu.sync_copy(nv_hbm.at[pl.ds(0, 16)], nv_v)
    nv = nv_v[...][0]

    @pl.when(base < nv)
    def _():
        pltpu.sync_copy(gtok_hbm.at[pl.ds(base, GROWS)], idx_v)
        bufs = (rows_a, rows_b)
        sems = (sem_a, sem_b)

        def fire(c):
            return pltpu.async_copy(
                hs_hbm.at[idx_v.at[pl.ds(c * GCH, GCH)]],
                bufs[c % 2], sems[c % 2])

        h = fire(0)
        for c in range(NGC):
            h_next = fire(c + 1) if c + 1 < NGC else None
            h.wait()
            pltpu.sync_copy(bufs[c % 2], out_hbm.at[pl.ds(base + c * GCH, GCH)])
            h = h_next


def _sc_gather_call(hs, gather_tok, nv_arr):
    mesh = plsc.VectorSubcoreMesh(core_axis_name="c", subcore_axis_name="s",
                                  num_cores=NC, num_subcores=NS)
    f = functools.partial(
        pl.kernel, _sc_gather_body, mesh=mesh,
        out_type=jax.ShapeDtypeStruct((PADDED, H), jnp.float32),
        scratch_types=[
            pltpu.VMEM((GROWS,), jnp.int32),
            pltpu.VMEM((GCH, H), jnp.float32),
            pltpu.VMEM((GCH, H), jnp.float32),
            pltpu.VMEM((16,), jnp.int32),
            pltpu.SemaphoreType.DMA,
            pltpu.SemaphoreType.DMA,
        ],
        name="sc_dispatch_gather",
    )()
    return f(hs, gather_tok, nv_arr)


def _ffn_body(nvb_ref, be_ref, bxi_ref, x_ref, wg_ref, wu_ref, wd_ref,
              wrow_ref, o_ref):
    b = pl.program_id(0)

    @pl.when(b < nvb_ref[0])
    def _():
        x = x_ref[...]
        a = lax.dot_general(x, wg_ref[0], (((1,), (0,)), ((), ())),
                            preferred_element_type=jnp.float32)
        u = lax.dot_general(x, wu_ref[0], (((1,), (0,)), ((), ())),
                            preferred_element_type=jnp.float32)
        hblk = (a * jax.nn.sigmoid(a)) * u * wrow_ref[0]
        o_ref[...] = lax.dot_general(hblk, wd_ref[0], (((1,), (0,)), ((), ())),
                                     preferred_element_type=jnp.float32)


def _ffn_call(x_sorted, w_gate, w_up, w_down, w_pad3, nvb, be, bxi):
    grid_spec = pltpu.PrefetchScalarGridSpec(
        num_scalar_prefetch=3,
        grid=(NB,),
        in_specs=[
            pl.BlockSpec((BM, H), lambda b, nvb, be, bxi: (bxi[b], 0)),
            pl.BlockSpec((1, H, FF), lambda b, nvb, be, bxi: (be[b], 0, 0)),
            pl.BlockSpec((1, H, FF), lambda b, nvb, be, bxi: (be[b], 0, 0)),
            pl.BlockSpec((1, FF, H), lambda b, nvb, be, bxi: (be[b], 0, 0)),
            pl.BlockSpec((1, BM, 1), lambda b, nvb, be, bxi: (b, 0, 0)),
        ],
        out_specs=pl.BlockSpec((BM, H), lambda b, nvb, be, bxi: (b, 0)),
    )
    return pl.pallas_call(
        _ffn_body,
        grid_spec=grid_spec,
        out_shape=jax.ShapeDtypeStruct((PADDED, H), jnp.float32),
    )(nvb, be, bxi, x_sorted, w_gate, w_up, w_down, w_pad3)


def _sc_combine_body(osort_hbm, inv_hbm, pairs_hbm,
                     idx_v, rows_a, rows_b, sem_a, sem_b):
    wid = lax.axis_index("s") * NC + lax.axis_index("c")
    base = wid * TPW * TOPK
    rows = TPW * TOPK
    pltpu.sync_copy(inv_hbm.at[pl.ds(base, rows)], idx_v)
    bufs = (rows_a, rows_b)
    sems = (sem_a, sem_b)
    rch = CPAIR * TOPK

    def fire(c):
        return pltpu.async_copy(
            osort_hbm.at[idx_v.at[pl.ds(c * rch, rch)]],
            bufs[c % 2], sems[c % 2])

    h = fire(0)
    for c in range(NCC):
        h_next = fire(c + 1) if c + 1 < NCC else None
        h.wait()
        pltpu.sync_copy(bufs[c % 2], pairs_hbm.at[pl.ds(base + c * rch, rch)])
        h = h_next


def _sc_combine_call(osort, inv):
    mesh = plsc.VectorSubcoreMesh(core_axis_name="c", subcore_axis_name="s",
                                  num_cores=NC, num_subcores=NS)
    f = functools.partial(
        pl.kernel, _sc_combine_body, mesh=mesh,
        out_type=jax.ShapeDtypeStruct((S * TOPK, H), jnp.float32),
        scratch_types=[
            pltpu.VMEM((TPW * TOPK,), jnp.int32),
            pltpu.VMEM((CPAIR * TOPK, H), jnp.float32),
            pltpu.VMEM((CPAIR * TOPK, H), jnp.float32),
            pltpu.SemaphoreType.DMA,
            pltpu.SemaphoreType.DMA,
        ],
        name="sc_combine_gather",
    )()
    return f(osort, inv)


def _add_body(a_ref, o_ref):
    o_ref[...] = a_ref[:, 0, :] + a_ref[:, 1, :]


def _add_call(pairs):
    return pl.pallas_call(
        _add_body,
        grid=(S // BG,),
        in_specs=[pl.BlockSpec((BG, TOPK, H), lambda i: (i, 0, 0))],
        out_specs=pl.BlockSpec((BG, H), lambda i: (i, 0)),
        out_shape=jax.ShapeDtypeStruct((S, H), jnp.float32),
    )(pairs.reshape(S, TOPK, H))


def kernel(hidden_states, gate_weight, gate_bias, w_gate, w_up, w_down):
    B_, S_, H_ = hidden_states.shape
    hs = hidden_states.reshape(S_, H_)
    topk_idx, topk_w = _gate_call(hs, gate_weight, gate_bias)
    (gather_tok, w_pad3, be, bxi, nvb, nv_arr, inv) = _routing(
        topk_idx, topk_w)
    x_sorted = _sc_gather_call(hs, gather_tok, nv_arr)
    osort = _ffn_call(x_sorted, w_gate, w_up, w_down, w_pad3, nvb, be, bxi)
    pairs = _sc_combine_call(osort, inv)
    final = _add_call(pairs)
    return final.reshape(B_, S_, H_)
